# baseline (speedup 1.0000x reference)
import jax
import jax.numpy as jnp
from jax.experimental import pallas as pl
from jax.experimental.pallas import tpu as pltpu

BSZ, N_HEADS, MAX_SEQ_LEN, HEAD_DIM = 8, 16, 4096, 64
SEQLEN = 16
BH = BSZ * N_HEADS
ROWS = MAX_SEQ_LEN * HEAD_DIM // 128
UROWS = SEQLEN * HEAD_DIM // 128
CB = 2
NCHUNK = BH // CB
NBUF = 8


def _in_copy(i, c_ref, buf, insem):
    return pltpu.make_async_copy(
        c_ref.at[pl.ds(i * CB, CB)], buf.at[i % NBUF], insem.at[i % NBUF]
    )


def _out_copy(i, o_ref, buf, outsem):
    return pltpu.make_async_copy(
        buf.at[i % NBUF], o_ref.at[pl.ds(i * CB, CB)], outsem.at[i % NBUF]
    )


def _body(pos_ref, c_ref, u_ref, o_ref, buf, ubuf, insem, outsem, usem):
    uin = pltpu.make_async_copy(u_ref, ubuf, usem)
    uin.start()
    for i in range(NBUF):
        _in_copy(i, c_ref, buf, insem).start()
    uin.wait()

    p = pos_ref[0]
    r0 = p // 2

    for i in range(NCHUNK):
        if i >= NBUF:
            _out_copy(i - NBUF, o_ref, buf, outsem).wait()
            _in_copy(i, c_ref, buf, insem).start()
        _in_copy(i, c_ref, buf, insem).wait()
        b = buf.at[i % NBUF]
        uv = ubuf[pl.ds(i * CB, CB), :, :]

        @pl.when(p % 2 == 0)
        def _even():
            b[:, pl.ds(r0, UROWS), :] = uv

        @pl.when(p % 2 == 1)
        def _odd():
            b[:, pl.ds(r0, UROWS), 64:128] = uv[:, :, 0:64]
            b[:, pl.ds(r0 + 1, UROWS), 0:64] = uv[:, :, 64:128]

        _out_copy(i, o_ref, buf, outsem).start()
    for i in range(NCHUNK - NBUF, NCHUNK):
        _out_copy(i, o_ref, buf, outsem).wait()


def kernel(cache, update, pos):
    c3 = cache.reshape(BH, ROWS, 128)
    u3 = update.reshape(BH, UROWS, 128)
    out = pl.pallas_call(
        _body,
        grid_spec=pltpu.PrefetchScalarGridSpec(
            num_scalar_prefetch=1,
            grid=(1,),
            in_specs=[
                pl.BlockSpec(memory_space=pl.ANY),
                pl.BlockSpec(memory_space=pl.ANY),
            ],
            out_specs=pl.BlockSpec(memory_space=pl.ANY),
            scratch_shapes=[
                pltpu.VMEM((NBUF, CB, ROWS, 128), jnp.float32),
                pltpu.VMEM((BH, UROWS, 128), jnp.float32),
                pltpu.SemaphoreType.DMA((NBUF,)),
                pltpu.SemaphoreType.DMA((NBUF,)),
                pltpu.SemaphoreType.DMA,
            ],
        ),
        out_shape=jax.ShapeDtypeStruct((BH, ROWS, 128), cache.dtype),
    )(pos, c3, u3)
    return out.reshape(BSZ, N_HEADS, MAX_SEQ_LEN, HEAD_DIM)


# speedup vs baseline: 1.8536x; 1.8536x over previous
import jax
import jax.numpy as jnp
from jax import lax
from jax.experimental import pallas as pl
from jax.experimental.pallas import tpu as pltpu
from jax.experimental.pallas import tpu_sc as plsc

BSZ, N_HEADS, MAX_SEQ_LEN, HEAD_DIM = 8, 16, 4096, 64
SEQLEN = 16
BH = BSZ * N_HEADS
NW = 32
ROWS_W = BH * MAX_SEQ_LEN // NW
CH = 512
NCH = ROWS_W // CH
NBUF = 2


def _in_copy(i, w, c_ref, buf, insem):
    return pltpu.make_async_copy(
        c_ref.at[pl.ds(w * ROWS_W + i * CH, CH)],
        buf.at[i % NBUF],
        insem.at[i % NBUF],
    )


def _out_copy(i, w, o_ref, buf, outsem):
    return pltpu.make_async_copy(
        buf.at[i % NBUF],
        o_ref.at[pl.ds(w * ROWS_W + i * CH, CH)],
        outsem.at[i % NBUF],
    )


def _sc_body(c_ref, o_ref, buf, insem, outsem):
    w = lax.axis_index("s") * 2 + lax.axis_index("c")
    for i in range(NBUF):
        _in_copy(i, w, c_ref, buf, insem).start()
    for i in range(NCH):
        _in_copy(i, w, c_ref, buf, insem).wait()
        _out_copy(i, w, o_ref, buf, outsem).start()
        nxt = i + NBUF
        if nxt < NCH:
            _out_copy(i, w, o_ref, buf, outsem).wait()
            _in_copy(nxt, w, c_ref, buf, insem).start()
    for i in range(NCH - NBUF, NCH):
        _out_copy(i, w, o_ref, buf, outsem).wait()


def _sc_copy(c2):
    mesh = plsc.VectorSubcoreMesh(core_axis_name="c", subcore_axis_name="s")
    return pl.kernel(
        _sc_body,
        mesh=mesh,
        out_type=jax.ShapeDtypeStruct((BH * MAX_SEQ_LEN, HEAD_DIM), jnp.float32),
        scratch_types=[
            pltpu.VMEM((NBUF, CH, HEAD_DIM), jnp.float32),
            pltpu.SemaphoreType.DMA((NBUF,)),
            pltpu.SemaphoreType.DMA((NBUF,)),
        ],
    )(c2)


def _upd_body(pos_ref, prev_ref, u_ref, o_ref, sem):
    del prev_ref
    p = pos_ref[0]
    cp = pltpu.make_async_copy(u_ref, o_ref.at[:, pl.ds(p, SEQLEN), :], sem)
    cp.start()
    cp.wait()


def _scatter_update(copied, u3, pos):
    return pl.pallas_call(
        _upd_body,
        grid_spec=pltpu.PrefetchScalarGridSpec(
            num_scalar_prefetch=1,
            grid=(1,),
            in_specs=[
                pl.BlockSpec(memory_space=pl.ANY),
                pl.BlockSpec((BH, SEQLEN, HEAD_DIM), lambda i, p: (0, 0, 0)),
            ],
            out_specs=pl.BlockSpec(memory_space=pl.ANY),
            scratch_shapes=[pltpu.SemaphoreType.DMA],
        ),
        out_shape=jax.ShapeDtypeStruct((BH, MAX_SEQ_LEN, HEAD_DIM), jnp.float32),
        input_output_aliases={1: 0},
    )(pos, copied, u3)


def kernel(cache, update, pos):
    c2 = cache.reshape(BH * MAX_SEQ_LEN, HEAD_DIM)
    u3 = update.reshape(BH, SEQLEN, HEAD_DIM)
    copied = _sc_copy(c2).reshape(BH, MAX_SEQ_LEN, HEAD_DIM)
    out = _scatter_update(copied, u3, pos)
    return out.reshape(BSZ, N_HEADS, MAX_SEQ_LEN, HEAD_DIM)
